# baseline (speedup 1.0000x reference)
import jax
import jax.numpy as jnp
from jax.experimental import pallas as pl
from jax.experimental.pallas import tpu as pltpu

_VMEM = 48 * 1024 * 1024


def _conv_stack_kernel(x_ref, w1_ref, b1_ref, w2_ref, b2_ref, w3_ref, b3_ref,
                       s1_ref, s2_ref, s3_ref, o_ref,
                       xs1, xs2, xs3, xs4, p1a, p1b, a20, a21, a22,
                       p2a, p2b, a30, a31, a32, p3a, p3b):
    w1 = w1_ref[...]; b1 = b1_ref[...]; sel1 = s1_ref[...]
    w2 = w2_ref[...]; b2 = b2_ref[...]; sel2 = s2_ref[...]
    w3 = w3_ref[...]; b3 = b3_ref[...]; sel3 = s3_ref[...]

    L = 100096
    for kw, xs in ((1, xs1), (2, xs2), (3, xs3), (4, xs4)):
        xs[:, :] = x_ref[0, :, kw:kw + L]

    for a2 in (a20, a21, a22):
        a2[:, 0:384] = jnp.zeros((8, 384), jnp.float32)
        a2[:, 129 * 384:131 * 384] = jnp.zeros((8, 768), jnp.float32)
    for a3 in (a30, a31, a32):
        a3[:, 0:256] = jnp.zeros((16, 256), jnp.float32)
        a3[:, 65 * 256:67 * 256] = jnp.zeros((16, 512), jnp.float32)

    for c in range(32):
        p1 = p1a if c % 2 == 0 else p1b
        for kh in range(5):
            for kw in range(5):
                st = (c * 8 + kh) * 384
                src = (x_ref[0, :, st:st + 3072] if kw == 0 else
                       (xs1, xs2, xs3, xs4)[kw - 1][:, st:st + 3072])
                p1[(kh * 5 + kw) * 8:(kh * 5 + kw + 1) * 8, :] = src
        acc = jnp.dot(w1, p1[...], preferred_element_type=jnp.float32)
        acc = jnp.maximum(acc + b1, 0.0)
        for i in range(4):
            ra, rb = (2 * i) * 384, (2 * i + 1) * 384
            m = jnp.maximum(
                jnp.maximum(acc[:, ra:ra + 256], acc[:, ra + 1:ra + 257]),
                jnp.maximum(acc[:, rb:rb + 256], acc[:, rb + 1:rb + 257]))
            t = jnp.dot(m, sel1, preferred_element_type=jnp.float32)
            r2 = c * 4 + i
            for kw, a2 in ((0, a20), (1, a21), (2, a22)):
                a2[:, (r2 + 1) * 384:(r2 + 2) * 384] = \
                    t[:, kw * 384:(kw + 1) * 384]

    for c in range(8):
        p2 = p2a if c % 2 == 0 else p2b
        for kh in range(3):
            for kw, a2 in ((0, a20), (1, a21), (2, a22)):
                st = (c * 16 + kh) * 384 + 128
                p2[(kh * 3 + kw) * 8:(kh * 3 + kw + 1) * 8, :] = \
                    a2[:, st:st + 6144]
        acc = jnp.dot(w2, p2[...], preferred_element_type=jnp.float32)
        acc = jnp.maximum(acc + b2, 0.0)
        for i in range(8):
            ra, rb = (2 * i) * 384, (2 * i + 1) * 384
            m = jnp.maximum(
                jnp.maximum(acc[:, ra:ra + 128], acc[:, ra + 1:ra + 129]),
                jnp.maximum(acc[:, rb:rb + 128], acc[:, rb + 1:rb + 129]))
            t = jnp.dot(m, sel2, preferred_element_type=jnp.float32)
            r3 = c * 8 + i
            for kw, a3 in ((0, a30), (1, a31), (2, a32)):
                a3[:, (r3 + 1) * 256:(r3 + 2) * 256] = \
                    t[:, kw * 256:(kw + 1) * 256]

    for c in range(4):
        p3 = p3a if c % 2 == 0 else p3b
        for kh in range(3):
            for kw, a3 in ((0, a30), (1, a31), (2, a32)):
                st = (c * 16 + kh) * 256 + 128
                p3[(kh * 3 + kw) * 16:(kh * 3 + kw + 1) * 16, :] = \
                    a3[:, st:st + 4096]
        acc = jnp.dot(w3, p3[...], preferred_element_type=jnp.float32)
        acc = jnp.maximum(acc + b3, 0.0)
        for g in range(2):
            parts = []
            for j in range(4):
                i = g * 4 + j
                ra, rb = (2 * i) * 256, (2 * i + 1) * 256
                m = jnp.maximum(
                    jnp.maximum(acc[:, ra:ra + 64], acc[:, ra + 1:ra + 65]),
                    jnp.maximum(acc[:, rb:rb + 64], acc[:, rb + 1:rb + 65]))
                parts.append(
                    jnp.dot(m, sel3[j], preferred_element_type=jnp.float32))
            gg = c * 2 + g
            o_ref[0, :, gg * 128:(gg + 1) * 128] = \
                parts[0] + parts[1] + parts[2] + parts[3]


def _conv_stack(xg, w1, b1, w2, b2, w3, b3):
    B = xg.shape[0]
    i2 = jnp.arange(0, 256, 2)

    def place(w, wp, nkw):
        cols = jnp.arange(wp)[None, :]
        rows = jnp.arange(w)[:, None]
        blocks = [(rows == 2 * (cols - (129 - kw))).astype(jnp.float32)
                  for kw in range(nkw)]
        return jnp.concatenate(blocks, axis=1)

    sel1 = place(256, 384, 3)
    sel2 = place(128, 256, 3)
    jj = jnp.arange(128)[None, :]
    sel3 = jnp.stack([
        (jnp.arange(64)[:, None] == 2 * (jj - 32 * q)).astype(jnp.float32) *
        ((jj >= 32 * q) & (jj < 32 * (q + 1))).astype(jnp.float32)
        for q in range(4)])

    consts = lambda b: (0, 0)
    out = pl.pallas_call(
        _conv_stack_kernel,
        out_shape=jax.ShapeDtypeStruct((B, 32, 1024), jnp.float32),
        grid_spec=pltpu.PrefetchScalarGridSpec(
            num_scalar_prefetch=0,
            grid=(B,),
            in_specs=[
                pl.BlockSpec((1, 8, 261 * 384), lambda b: (b, 0, 0)),
                pl.BlockSpec((8, 200), consts),
                pl.BlockSpec((8, 1), consts),
                pl.BlockSpec((16, 72), consts),
                pl.BlockSpec((16, 1), consts),
                pl.BlockSpec((32, 144), consts),
                pl.BlockSpec((32, 1), consts),
                pl.BlockSpec((256, 3 * 384), consts),
                pl.BlockSpec((128, 3 * 256), consts),
                pl.BlockSpec((4, 64, 128), lambda b: (0, 0, 0)),
            ],
            out_specs=pl.BlockSpec((1, 32, 1024), lambda b: (b, 0, 0)),
            scratch_shapes=[
                pltpu.VMEM((8, 100096), jnp.float32),
                pltpu.VMEM((8, 100096), jnp.float32),
                pltpu.VMEM((8, 100096), jnp.float32),
                pltpu.VMEM((8, 100096), jnp.float32),
                pltpu.VMEM((200, 3072), jnp.float32),
                pltpu.VMEM((200, 3072), jnp.float32),
                pltpu.VMEM((8, 131 * 384), jnp.float32),
                pltpu.VMEM((8, 131 * 384), jnp.float32),
                pltpu.VMEM((8, 131 * 384), jnp.float32),
                pltpu.VMEM((72, 6144), jnp.float32),
                pltpu.VMEM((72, 6144), jnp.float32),
                pltpu.VMEM((16, 67 * 256), jnp.float32),
                pltpu.VMEM((16, 67 * 256), jnp.float32),
                pltpu.VMEM((16, 67 * 256), jnp.float32),
                pltpu.VMEM((144, 4096), jnp.float32),
                pltpu.VMEM((144, 4096), jnp.float32),
            ],
        ),
        compiler_params=pltpu.CompilerParams(
            dimension_semantics=("parallel",),
            vmem_limit_bytes=_VMEM),
    )(xg, w1, b1, w2, b2, w3, b3, sel1, sel2, sel3)
    return out.reshape(B, 32 * 1024)


def _fc1_kernel(x_ref, w_ref, b_ref, o_ref, acc_ref):
    @pl.when(pl.program_id(1) == 0)
    def _():
        acc_ref[...] = jnp.zeros_like(acc_ref)

    acc_ref[...] += jnp.dot(x_ref[...], w_ref[...],
                            preferred_element_type=jnp.float32)

    @pl.when(pl.program_id(1) == pl.num_programs(1) - 1)
    def _():
        o_ref[...] = jnp.maximum(acc_ref[...] + b_ref[...], 0.0)


def _fc1(x, w, b):
    M, K = x.shape
    Np = w.shape[1]
    NB, TK = 2, 4096
    Nb = Np // NB
    return pl.pallas_call(
        _fc1_kernel,
        out_shape=jax.ShapeDtypeStruct((M, Np), jnp.float32),
        grid_spec=pltpu.PrefetchScalarGridSpec(
            num_scalar_prefetch=0,
            grid=(NB, K // TK),
            in_specs=[
                pl.BlockSpec((M, TK), lambda n, k: (0, k)),
                pl.BlockSpec((TK, Nb), lambda n, k: (k, n)),
                pl.BlockSpec((1, Nb), lambda n, k: (0, n)),
            ],
            out_specs=pl.BlockSpec((M, Nb), lambda n, k: (0, n)),
            scratch_shapes=[pltpu.VMEM((M, Nb), jnp.float32)],
        ),
        compiler_params=pltpu.CompilerParams(
            dimension_semantics=("parallel", "arbitrary"),
            vmem_limit_bytes=_VMEM),
    )(x, w, b)


def _head_kernel(x_ref, w2_ref, b2_ref, w3_ref, b3_ref, w4_ref, b4_ref,
                 w5_ref, b5_ref, o_ref):
    y = x_ref[...]
    for w_r, b_r in ((w2_ref, b2_ref), (w3_ref, b3_ref),
                     (w4_ref, b4_ref), (w5_ref, b5_ref)):
        y = jnp.maximum(
            jnp.dot(y, w_r[...], preferred_element_type=jnp.float32)
            + b_r[...], 0.0)
    o_ref[...] = y


def _head(x, *wbs):
    M = x.shape[0]
    args = (x,) + tuple(wbs)
    return pl.pallas_call(
        _head_kernel,
        out_shape=jax.ShapeDtypeStruct((M, wbs[-2].shape[1]), jnp.float32),
        in_specs=[pl.BlockSpec(a.shape, lambda: (0, 0)) for a in args],
        out_specs=pl.BlockSpec((M, wbs[-2].shape[1]), lambda: (0, 0)),
        compiler_params=pltpu.CompilerParams(vmem_limit_bytes=_VMEM),
    )(*args)


def kernel(x, w1, b1, w2, b2, w3, b3, fw1, fb1, fw2, fb2, fw3, fb3,
           fw4, fb4, fw5, fb5):
    B, C = x.shape[:2]
    xg = jnp.pad(x, ((0, 0), (0, 8 - C), (2, 3), (2, 126))
                 ).reshape(B, 8, 261 * 384)
    feat = _conv_stack(xg, w1, b1, w2, b2, w3, b3)
    y1 = _fc1(feat, fw1, fb1)
    y = _head(y1[:, :1200], fw2, fb2, fw3, fb3, fw4, fb4, fw5, fb5)
    return feat, y
